# baseline (speedup 1.0000x reference)
import jax
import jax.numpy as jnp
from jax import lax
from jax.experimental import pallas as pl
from jax.experimental.pallas import tpu as pltpu
from jax.experimental.pallas import tpu_sc as plsc

N, D = 10000, 128
G = 4
LAMB = 0.7
CH = 80
NCHUNK = N // CH
NW = 32
NITER = (NCHUNK + NW - 1) // NW


def _mix_body(x_hbm, xp_hbm, idx_hbm, out_hbm,
              idx_v, xa0, xa1, xb0, xb1, o0, o1,
              isem, dsem0, dsem1, ssem0, ssem1):
    info = plsc.get_sparse_core_info()
    wid = lax.axis_index("s") * info.num_cores + lax.axis_index("c")

    xa = (xa0, xa1)
    xb = (xb0, xb1)
    ov = (o0, o1)
    dsem = (dsem0, dsem1)
    ssem = (ssem0, ssem1)

    last = NCHUNK - 1
    base = [None] * NITER
    icopy = [None] * NITER
    for i in range(NITER):
        c = jnp.minimum(wid + i * NW, last)
        base[i] = c * CH
        icopy[i] = pltpu.async_copy(
            idx_hbm.at[pl.ds(base[i], CH)], idx_v.at[i], isem)

    gcopy = [None] * NITER
    lcopy = [None] * NITER
    scopy = [None] * NITER

    def launch(i):
        b = i % 2
        icopy[i].wait()
        gcopy[i] = pltpu.async_copy(x_hbm.at[idx_v.at[i]], xb[b], dsem[b])
        lcopy[i] = pltpu.async_copy(xp_hbm.at[pl.ds(base[i], CH)], xa[b],
                                    dsem[b])

    launch(0)
    for i in range(NITER):
        b = i % 2
        if i + 1 < NITER:
            if i - 1 >= 0:
                scopy[i - 1].wait()
            launch(i + 1)
        gcopy[i].wait()
        lcopy[i].wait()

        def row_body(r, rcarry):
            for g in range(G):
                wa = xa[b][r, g]
                alo = lax.slice(wa, (0,), (16,)).astype(jnp.float32)
                ahi = lax.slice(wa, (16,), (32,)).astype(jnp.float32)
                s0 = pl.ds(g * 32, 16)
                s1 = pl.ds(g * 32 + 16, 16)
                ov[b][r, s0] = LAMB * alo + (1.0 - LAMB) * xb[b][r, s0]
                ov[b][r, s1] = LAMB * ahi + (1.0 - LAMB) * xb[b][r, s1]
            return rcarry

        lax.fori_loop(0, CH, row_body, 0, unroll=False)
        scopy[i] = pltpu.async_copy(ov[b], out_hbm.at[pl.ds(base[i], CH)],
                                    ssem[b])

    scopy[NITER - 2].wait()
    scopy[NITER - 1].wait()


@jax.jit
def _mix(x, xp, idx32):
    mesh = plsc.VectorSubcoreMesh(core_axis_name="c", subcore_axis_name="s")
    f = pl.kernel(
        _mix_body,
        mesh=mesh,
        out_type=jax.ShapeDtypeStruct((N, D), jnp.float32),
        scratch_types=[
            pltpu.VMEM((NITER, CH), jnp.int32),
            pltpu.VMEM((CH, G, 32), jnp.bfloat16),
            pltpu.VMEM((CH, G, 32), jnp.bfloat16),
            pltpu.VMEM((CH, D), jnp.float32),
            pltpu.VMEM((CH, D), jnp.float32),
            pltpu.VMEM((CH, D), jnp.float32),
            pltpu.VMEM((CH, D), jnp.float32),
            pltpu.SemaphoreType.DMA,
            pltpu.SemaphoreType.DMA,
            pltpu.SemaphoreType.DMA,
            pltpu.SemaphoreType.DMA,
            pltpu.SemaphoreType.DMA,
        ],
    )
    return f(x, xp, idx32)


def kernel(x, y, edge_index, train_mask, test_mask, pair_idx):
    xp = x.astype(jnp.bfloat16).reshape(N, G, 32)
    x_mix = _mix(x, xp, pair_idx.astype(jnp.int32))
    new_y = y.astype(jnp.int32)
    return (x_mix, new_y, edge_index, train_mask, test_mask)


# speedup vs baseline: 1.6076x; 1.6076x over previous
import jax
import jax.numpy as jnp
from jax import lax
from jax.experimental import pallas as pl
from jax.experimental.pallas import tpu as pltpu
from jax.experimental.pallas import tpu_sc as plsc

N, D = 10000, 128
LAMB = 0.7
CH = 80
NCHUNK = N // CH
NW = 32
NITER = (NCHUNK + NW - 1) // NW


def _mix_body(x_hbm, idx_hbm, y_hbm, tm_hbm, sm_hbm,
              out_hbm, y_out, tm_out, sm_out,
              idx_v, xa0, xa1, xb0, xb1, yv, mv,
              isem, dsem0, dsem1, ssem0, ssem1, psem):
    info = plsc.get_sparse_core_info()
    wid = lax.axis_index("s") * info.num_cores + lax.axis_index("c")

    xa = (xa0, xa1)
    xb = (xb0, xb1)
    dsem = (dsem0, dsem1)
    ssem = (ssem0, ssem1)

    @pl.when(wid == 1)
    def _():
        pltpu.async_copy(y_hbm, yv, psem).wait()
        pltpu.async_copy(yv, y_out, psem).wait()

    @pl.when(wid == 2)
    def _():
        pltpu.async_copy(tm_hbm, mv, psem).wait()
        pltpu.async_copy(mv, tm_out, psem).wait()

    @pl.when(wid == 3)
    def _():
        pltpu.async_copy(sm_hbm, mv, psem).wait()
        pltpu.async_copy(mv, sm_out, psem).wait()

    last = NCHUNK - 1
    base = [None] * NITER
    icopy = [None] * NITER
    for i in range(NITER):
        c = jnp.minimum(wid + i * NW, last)
        base[i] = c * CH
        icopy[i] = pltpu.async_copy(
            idx_hbm.at[pl.ds(base[i], CH)], idx_v.at[i], isem)

    gcopy = [None] * NITER
    lcopy = [None] * NITER
    scopy = [None] * NITER

    def launch(i):
        b = i % 2
        icopy[i].wait()
        gcopy[i] = pltpu.async_copy(x_hbm.at[idx_v.at[i]], xb[b], dsem[b])
        lcopy[i] = pltpu.async_copy(x_hbm.at[pl.ds(base[i], CH)], xa[b],
                                    dsem[b])

    launch(0)
    for i in range(NITER):
        b = i % 2
        if i + 1 < NITER:
            if i - 1 >= 0:
                scopy[i - 1].wait()
            launch(i + 1)
        gcopy[i].wait()
        lcopy[i].wait()

        def row_body(r, rcarry):
            for cc in range(D // 16):
                s = pl.ds(cc * 16, 16)
                xa[b][r, s] = LAMB * xa[b][r, s] + (1.0 - LAMB) * xb[b][r, s]
            return rcarry

        lax.fori_loop(0, CH, row_body, 0, unroll=False)
        scopy[i] = pltpu.async_copy(xa[b], out_hbm.at[pl.ds(base[i], CH)],
                                    ssem[b])

    scopy[NITER - 2].wait()
    scopy[NITER - 1].wait()


@jax.jit
def _mix(x, idx32, y, tm, sm):
    mesh = plsc.VectorSubcoreMesh(core_axis_name="c", subcore_axis_name="s")
    f = pl.kernel(
        _mix_body,
        mesh=mesh,
        out_type=(
            jax.ShapeDtypeStruct((N, D), jnp.float32),
            jax.ShapeDtypeStruct((N,), jnp.int32),
            jax.ShapeDtypeStruct((N,), jnp.bool_),
            jax.ShapeDtypeStruct((N,), jnp.bool_),
        ),
        scratch_types=[
            pltpu.VMEM((NITER, CH), jnp.int32),
            pltpu.VMEM((CH, D), jnp.float32),
            pltpu.VMEM((CH, D), jnp.float32),
            pltpu.VMEM((CH, D), jnp.float32),
            pltpu.VMEM((CH, D), jnp.float32),
            pltpu.VMEM((N,), jnp.int32),
            pltpu.VMEM((N,), jnp.bool_),
            pltpu.SemaphoreType.DMA,
            pltpu.SemaphoreType.DMA,
            pltpu.SemaphoreType.DMA,
            pltpu.SemaphoreType.DMA,
            pltpu.SemaphoreType.DMA,
            pltpu.SemaphoreType.DMA,
        ],
    )
    return f(x, idx32, y, tm, sm)


def kernel(x, y, edge_index, train_mask, test_mask, pair_idx):
    x_mix, new_y, tm, sm = _mix(x, pair_idx.astype(jnp.int32),
                                y.astype(jnp.int32), train_mask, test_mask)
    return (x_mix, new_y, edge_index, tm, sm)


# speedup vs baseline: 1.6374x; 1.0185x over previous
import jax
import jax.numpy as jnp
from jax import lax
from jax.experimental import pallas as pl
from jax.experimental.pallas import tpu as pltpu
from jax.experimental.pallas import tpu_sc as plsc

N, D = 10000, 128
LAMB = 0.7
CH = 80
NCHUNK = N // CH
NW = 32
NITER = (NCHUNK + NW - 1) // NW


def _mix_body(x_hbm, idx_hbm, out_hbm,
              idx_v, xa0, xa1, xb0, xb1,
              isem, dsem0, dsem1, ssem0, ssem1):
    info = plsc.get_sparse_core_info()
    wid = lax.axis_index("s") * info.num_cores + lax.axis_index("c")

    xa = (xa0, xa1)
    xb = (xb0, xb1)
    dsem = (dsem0, dsem1)
    ssem = (ssem0, ssem1)

    last = NCHUNK - 1
    base = [None] * NITER
    icopy = [None] * NITER
    for i in range(NITER):
        c = jnp.minimum(wid + i * NW, last)
        base[i] = c * CH
        icopy[i] = pltpu.async_copy(
            idx_hbm.at[pl.ds(base[i], CH)], idx_v.at[i], isem)

    gcopy = [None] * NITER
    lcopy = [None] * NITER
    scopy = [None] * NITER

    def launch(i):
        b = i % 2
        icopy[i].wait()
        gcopy[i] = pltpu.async_copy(x_hbm.at[idx_v.at[i]], xb[b], dsem[b])
        lcopy[i] = pltpu.async_copy(x_hbm.at[pl.ds(base[i], CH)], xa[b],
                                    dsem[b])

    launch(0)
    for i in range(NITER):
        b = i % 2
        if i + 1 < NITER:
            if i - 1 >= 0:
                scopy[i - 1].wait()
            launch(i + 1)
        gcopy[i].wait()
        lcopy[i].wait()

        def row_body(r, rcarry):
            for cc in range(D // 16):
                s = pl.ds(cc * 16, 16)
                xa[b][r, s] = LAMB * xa[b][r, s] + (1.0 - LAMB) * xb[b][r, s]
            return rcarry

        lax.fori_loop(0, CH, row_body, 0, unroll=False)
        scopy[i] = pltpu.async_copy(xa[b], out_hbm.at[pl.ds(base[i], CH)],
                                    ssem[b])

    scopy[NITER - 2].wait()
    scopy[NITER - 1].wait()


@jax.jit
def _mix(x, idx32):
    mesh = plsc.VectorSubcoreMesh(core_axis_name="c", subcore_axis_name="s")
    f = pl.kernel(
        _mix_body,
        mesh=mesh,
        out_type=jax.ShapeDtypeStruct((N, D), jnp.float32),
        scratch_types=[
            pltpu.VMEM((NITER, CH), jnp.int32),
            pltpu.VMEM((CH, D), jnp.float32),
            pltpu.VMEM((CH, D), jnp.float32),
            pltpu.VMEM((CH, D), jnp.float32),
            pltpu.VMEM((CH, D), jnp.float32),
            pltpu.SemaphoreType.DMA,
            pltpu.SemaphoreType.DMA,
            pltpu.SemaphoreType.DMA,
            pltpu.SemaphoreType.DMA,
            pltpu.SemaphoreType.DMA,
        ],
    )
    return f(x, idx32)


def kernel(x, y, edge_index, train_mask, test_mask, pair_idx):
    x_mix = _mix(x, pair_idx.astype(jnp.int32))
    new_y = y.astype(jnp.int32)
    return (x_mix, new_y, edge_index, train_mask, test_mask)
